# Initial kernel scaffold; baseline (speedup 1.0000x reference)
import jax
import jax.numpy as jnp
from jax.experimental import pallas as pl


def kernel(x, A, t, W_emb, b_emb, W_gcn, b_gcn, W1, b1, W2, b2):
    raise NotImplementedError("write your pallas kernel here")


# speedup vs baseline: 2151.1748x; 2151.1748x over previous
import functools

import jax
import jax.numpy as jnp
from jax.experimental import pallas as pl

_BI = 256


def _emb_body(x_ref, we_ref, be_ref, wg_ref, h_ref, xw_ref):
    h = jnp.maximum(
        jnp.dot(x_ref[...], we_ref[...], preferred_element_type=jnp.float32)
        + be_ref[...],
        0.0,
    )
    h_ref[...] = h
    xw_ref[...] = jnp.dot(h, wg_ref[...], preferred_element_type=jnp.float32)


def _pass1_body(a_ref, hi_ref, h_ref, deg_ref, m_ref, a8_ref):
    a_blk = a_ref[...]
    mask = a_blk != 0
    s = jax.lax.dot_general(
        hi_ref[...], h_ref[...], (((1,), (1,)), ((), ())),
        preferred_element_type=jnp.float32, precision=jax.lax.Precision.HIGHEST,
    )
    blk_max = jnp.max(jnp.where(mask, s, -jnp.inf))
    blk_deg = jnp.sum(mask.astype(jnp.float32), axis=0, keepdims=True)
    a8_ref[...] = a_blk.astype(jnp.int8)

    @pl.when(pl.program_id(0) == 0)
    def _init():
        m_ref[...] = blk_max.reshape(1, 1)
        deg_ref[...] = blk_deg

    @pl.when(pl.program_id(0) != 0)
    def _acc():
        m_ref[...] = jnp.maximum(m_ref[...], blk_max.reshape(1, 1))
        deg_ref[...] += blk_deg


def _pass2_body(a8_ref, hi_ref, h_ref, xwi_ref, dinvi_ref, m_ref,
                deno_ref, u_ref):
    a8 = a8_ref[...]
    mask = a8 != 0
    s = jax.lax.dot_general(
        hi_ref[...], h_ref[...], (((1,), (1,)), ((), ())),
        preferred_element_type=jnp.float32, precision=jax.lax.Precision.HIGHEST,
    )
    e = jnp.where(mask, jnp.exp(s - m_ref[...]), 0.0)
    blk_deno = jnp.sum(e, axis=0, keepdims=True)
    y = dinvi_ref[...] * xwi_ref[...]
    blk_u = jax.lax.dot_general(
        a8.astype(jnp.float32), y, (((0,), (0,)), ((), ())),
        preferred_element_type=jnp.float32, precision=jax.lax.Precision.HIGHEST,
    )

    @pl.when(pl.program_id(0) == 0)
    def _init():
        deno_ref[...] = blk_deno
        u_ref[...] = blk_u

    @pl.when(pl.program_id(0) != 0)
    def _acc():
        deno_ref[...] += blk_deno
        u_ref[...] += blk_u


def _pass3_body(a8_ref, hi_ref, h_ref, wi_ref, m_ref, zs_ref):
    mask = a8_ref[...] != 0
    s = jax.lax.dot_general(
        hi_ref[...], h_ref[...], (((1,), (1,)), ((), ())),
        preferred_element_type=jnp.float32, precision=jax.lax.Precision.HIGHEST,
    )
    e = jnp.where(mask, jnp.exp(s - m_ref[...]), 0.0)
    blk_z = jnp.sum(wi_ref[...] * e, axis=0, keepdims=True)

    @pl.when(pl.program_id(0) == 0)
    def _init():
        zs_ref[...] = blk_z

    @pl.when(pl.program_id(0) != 0)
    def _acc():
        zs_ref[...] += blk_z


def _final_body(h_ref, xw_ref, u_ref, dinv_ref, t_ref, zs_ref, bg_ref,
                w1a_ref, w1t_ref, w1z_ref, b1_ref, w2_ref, b2_ref,
                out_ref, rep_ref, z_ref):
    dinv = dinv_ref[...]
    xw = xw_ref[...]
    agg = dinv * u_ref[...] + (dinv * dinv) * xw
    rep = h_ref[...] + jnp.maximum(agg + bg_ref[...], 0.0)
    rep_ref[...] = rep
    t_col = t_ref[...]
    z_col = t_col * zs_ref[...]
    z_ref[...] = z_col
    hid = jnp.maximum(
        jnp.dot(rep, w1a_ref[...], preferred_element_type=jnp.float32, precision=jax.lax.Precision.HIGHEST)
        + t_col * w1t_ref[...]
        + z_col * w1z_ref[...]
        + b1_ref[...],
        0.0,
    )
    out_ref[...] = (
        jnp.dot(hid, w2_ref[...], preferred_element_type=jnp.float32, precision=jax.lax.Precision.HIGHEST)
        + b2_ref[...]
    )


@functools.partial(jax.jit, static_argnames=())
def kernel(x, A, t, W_emb, b_emb, W_gcn, b_gcn, W1, b1, W2, b2):
    n, n_in = x.shape
    n_h = W_emb.shape[1]
    f32 = jnp.float32
    nblk = n // _BI

    h, xw = pl.pallas_call(
        _emb_body,
        out_shape=(
            jax.ShapeDtypeStruct((n, n_h), f32),
            jax.ShapeDtypeStruct((n, W_gcn.shape[1]), f32),
        ),
    )(x, W_emb, b_emb.reshape(1, -1), W_gcn)

    blk_i = pl.BlockSpec((_BI, n), lambda i: (i, 0))
    h_i = pl.BlockSpec((_BI, n_h), lambda i: (i, 0))
    h_full = pl.BlockSpec((n, n_h), lambda i: (0, 0))
    col_i = pl.BlockSpec((_BI, 1), lambda i: (i, 0))
    row_acc = pl.BlockSpec((1, n), lambda i: (0, 0))
    scalar = pl.BlockSpec((1, 1), lambda i: (0, 0))

    deg, m, a8 = pl.pallas_call(
        _pass1_body,
        grid=(nblk,),
        in_specs=[blk_i, h_i, h_full],
        out_specs=(row_acc, scalar, blk_i),
        out_shape=(
            jax.ShapeDtypeStruct((1, n), f32),
            jax.ShapeDtypeStruct((1, 1), f32),
            jax.ShapeDtypeStruct((n, n), jnp.int8),
        ),
    )(A, h, h)

    dinv = 1.0 / jnp.sqrt(deg + 1.0)
    dinv_col = dinv.reshape(n, 1)

    deno, u = pl.pallas_call(
        _pass2_body,
        grid=(nblk,),
        in_specs=[blk_i, h_i, h_full, h_i, col_i, scalar],
        out_specs=(row_acc, pl.BlockSpec((n, n_h), lambda i: (0, 0))),
        out_shape=(
            jax.ShapeDtypeStruct((1, n), f32),
            jax.ShapeDtypeStruct((n, n_h), f32),
        ),
    )(a8, h, h, xw, dinv_col, m)

    w_col = (1.0 / (deno + 1e-8)).reshape(n, 1)

    zs = pl.pallas_call(
        _pass3_body,
        grid=(nblk,),
        in_specs=[blk_i, h_i, h_full, col_i, scalar],
        out_specs=row_acc,
        out_shape=jax.ShapeDtypeStruct((1, n), f32),
    )(a8, h, h, w_col, m)

    out, rep, z_col = pl.pallas_call(
        _final_body,
        out_shape=(
            jax.ShapeDtypeStruct((n, 1), f32),
            jax.ShapeDtypeStruct((n, n_h), f32),
            jax.ShapeDtypeStruct((n, 1), f32),
        ),
    )(h, xw, u, dinv_col, t.reshape(n, 1), zs.reshape(n, 1),
      b_gcn.reshape(1, -1), W1[:n_h], W1[n_h:n_h + 1], W1[n_h + 1:n_h + 2],
      b1.reshape(1, -1), W2, b2.reshape(1, 1))

    return out, rep, z_col.reshape(n)


# speedup vs baseline: 2486.2605x; 1.1558x over previous
import functools

import jax
import jax.numpy as jnp
from jax.experimental import pallas as pl

_BI = 256


def _emb_body(x_ref, we_ref, be_ref, wg_ref, h_ref, xw_ref):
    h = jnp.maximum(
        jnp.dot(x_ref[...], we_ref[...], preferred_element_type=jnp.float32)
        + be_ref[...],
        0.0,
    )
    h_ref[...] = h
    xw_ref[...] = jnp.dot(h, wg_ref[...], preferred_element_type=jnp.float32)


def _pass1_body(a_ref, hi_ref, h_ref, deg_ref, m_ref, a8_ref):
    a_blk = a_ref[...]
    mask = a_blk != 0
    s = jax.lax.dot_general(
        hi_ref[...], h_ref[...], (((1,), (1,)), ((), ())),
        preferred_element_type=jnp.float32, precision=jax.lax.Precision.HIGHEST,
    )
    blk_max = jnp.max(jnp.where(mask, s, -jnp.inf))
    blk_deg = jnp.sum(mask.astype(jnp.float32), axis=0, keepdims=True)
    a8_ref[...] = a_blk.astype(jnp.int8)

    @pl.when(pl.program_id(0) == 0)
    def _init():
        m_ref[...] = blk_max.reshape(1, 1)
        deg_ref[...] = blk_deg

    @pl.when(pl.program_id(0) != 0)
    def _acc():
        m_ref[...] = jnp.maximum(m_ref[...], blk_max.reshape(1, 1))
        deg_ref[...] += blk_deg


def _pass2_body(a8_ref, hi_ref, h_ref, xwi_ref, dinvi_ref, m_ref,
                deno_ref, u_ref):
    af = a8_ref[...].astype(jnp.float32)
    s = jax.lax.dot_general(
        hi_ref[...], h_ref[...], (((1,), (1,)), ((), ())),
        preferred_element_type=jnp.float32, precision=jax.lax.Precision.HIGHEST,
    )
    e = af * jnp.exp(s - m_ref[...])
    blk_deno = jnp.sum(e, axis=0, keepdims=True)
    y = dinvi_ref[...] * xwi_ref[...]
    blk_u = jax.lax.dot_general(
        af, y, (((0,), (0,)), ((), ())),
        preferred_element_type=jnp.float32,
    )

    @pl.when(pl.program_id(0) == 0)
    def _init():
        deno_ref[...] = blk_deno
        u_ref[...] = blk_u

    @pl.when(pl.program_id(0) != 0)
    def _acc():
        deno_ref[...] += blk_deno
        u_ref[...] += blk_u


def _pass3_body(a8_ref, hi_ref, h_ref, wi_ref, m_ref, zs_ref):
    af = a8_ref[...].astype(jnp.float32)
    s = jax.lax.dot_general(
        hi_ref[...], h_ref[...], (((1,), (1,)), ((), ())),
        preferred_element_type=jnp.float32, precision=jax.lax.Precision.HIGHEST,
    )
    e = af * jnp.exp(s - m_ref[...])
    blk_z = jnp.sum(wi_ref[...] * e, axis=0, keepdims=True)

    @pl.when(pl.program_id(0) == 0)
    def _init():
        zs_ref[...] = blk_z

    @pl.when(pl.program_id(0) != 0)
    def _acc():
        zs_ref[...] += blk_z


def _final_body(h_ref, xw_ref, u_ref, dinv_ref, t_ref, zs_ref, bg_ref,
                w1a_ref, w1t_ref, w1z_ref, b1_ref, w2_ref, b2_ref,
                out_ref, rep_ref, z_ref):
    dinv = dinv_ref[...]
    xw = xw_ref[...]
    agg = dinv * u_ref[...] + (dinv * dinv) * xw
    rep = h_ref[...] + jnp.maximum(agg + bg_ref[...], 0.0)
    rep_ref[...] = rep
    t_col = t_ref[...]
    z_col = t_col * zs_ref[...]
    z_ref[...] = z_col
    hid = jnp.maximum(
        jnp.dot(rep, w1a_ref[...], preferred_element_type=jnp.float32, precision=jax.lax.Precision.HIGHEST)
        + t_col * w1t_ref[...]
        + z_col * w1z_ref[...]
        + b1_ref[...],
        0.0,
    )
    out_ref[...] = (
        jnp.dot(hid, w2_ref[...], preferred_element_type=jnp.float32, precision=jax.lax.Precision.HIGHEST)
        + b2_ref[...]
    )


@functools.partial(jax.jit, static_argnames=())
def kernel(x, A, t, W_emb, b_emb, W_gcn, b_gcn, W1, b1, W2, b2):
    n, n_in = x.shape
    n_h = W_emb.shape[1]
    f32 = jnp.float32
    nblk = n // _BI

    h, xw = pl.pallas_call(
        _emb_body,
        out_shape=(
            jax.ShapeDtypeStruct((n, n_h), f32),
            jax.ShapeDtypeStruct((n, W_gcn.shape[1]), f32),
        ),
    )(x, W_emb, b_emb.reshape(1, -1), W_gcn)

    blk_i = pl.BlockSpec((_BI, n), lambda i: (i, 0))
    h_i = pl.BlockSpec((_BI, n_h), lambda i: (i, 0))
    h_full = pl.BlockSpec((n, n_h), lambda i: (0, 0))
    col_i = pl.BlockSpec((_BI, 1), lambda i: (i, 0))
    row_acc = pl.BlockSpec((1, n), lambda i: (0, 0))
    scalar = pl.BlockSpec((1, 1), lambda i: (0, 0))

    deg, m, a8 = pl.pallas_call(
        _pass1_body,
        grid=(nblk,),
        in_specs=[blk_i, h_i, h_full],
        out_specs=(row_acc, scalar, blk_i),
        out_shape=(
            jax.ShapeDtypeStruct((1, n), f32),
            jax.ShapeDtypeStruct((1, 1), f32),
            jax.ShapeDtypeStruct((n, n), jnp.int8),
        ),
    )(A, h, h)

    dinv = 1.0 / jnp.sqrt(deg + 1.0)
    dinv_col = dinv.reshape(n, 1)

    deno, u = pl.pallas_call(
        _pass2_body,
        grid=(nblk,),
        in_specs=[blk_i, h_i, h_full, h_i, col_i, scalar],
        out_specs=(row_acc, pl.BlockSpec((n, n_h), lambda i: (0, 0))),
        out_shape=(
            jax.ShapeDtypeStruct((1, n), f32),
            jax.ShapeDtypeStruct((n, n_h), f32),
        ),
    )(a8, h, h, xw, dinv_col, m)

    w_col = (1.0 / (deno + 1e-8)).reshape(n, 1)

    zs = pl.pallas_call(
        _pass3_body,
        grid=(nblk,),
        in_specs=[blk_i, h_i, h_full, col_i, scalar],
        out_specs=row_acc,
        out_shape=jax.ShapeDtypeStruct((1, n), f32),
    )(a8, h, h, w_col, m)

    out, rep, z_col = pl.pallas_call(
        _final_body,
        out_shape=(
            jax.ShapeDtypeStruct((n, 1), f32),
            jax.ShapeDtypeStruct((n, n_h), f32),
            jax.ShapeDtypeStruct((n, 1), f32),
        ),
    )(h, xw, u, dinv_col, t.reshape(n, 1), zs.reshape(n, 1),
      b_gcn.reshape(1, -1), W1[:n_h], W1[n_h:n_h + 1], W1[n_h + 1:n_h + 2],
      b1.reshape(1, -1), W2, b2.reshape(1, 1))

    return out, rep, z_col.reshape(n)


# speedup vs baseline: 3491.4901x; 1.4043x over previous
import functools

import jax
import jax.numpy as jnp
from jax.experimental import pallas as pl
from jax.experimental.pallas import tpu as pltpu

_BI = 256
_NB = 8
_HIGHEST = jax.lax.Precision.HIGHEST


def _mega_body(x_ref, a_ref, t_ref, we_ref, be_ref, wg_ref, bg_ref,
               w1a_ref, w1t_ref, w1z_ref, b1_ref, w2_ref, b2_ref,
               out_ref, rep_ref, z_ref,
               h_s, xw_s, a_s, e_s, m_s, deg_s, deno_s, u_s, zs_s):
    s = pl.program_id(0)

    @pl.when(s == 0)
    def _p0():
        h = jnp.maximum(
            jnp.dot(x_ref[...], we_ref[...],
                    preferred_element_type=jnp.float32) + be_ref[...],
            0.0,
        )
        h_s[...] = h
        xw_s[...] = jnp.dot(h, wg_ref[...],
                            preferred_element_type=jnp.float32)

    @pl.when((s >= 1) & (s <= _NB))
    def _p1():
        i = s - 1
        a_blk = a_ref[...]
        af = a_blk.astype(jnp.float32)
        hi = h_s[pl.ds(i * _BI, _BI), :]
        sblk = jax.lax.dot_general(
            hi, h_s[...], (((1,), (1,)), ((), ())),
            preferred_element_type=jnp.float32, precision=_HIGHEST,
        )
        blk_max = jnp.max(jnp.where(a_blk != 0, sblk, -jnp.inf)).reshape(1, 1)
        blk_deg = jnp.sum(af, axis=0, keepdims=True)
        a_s[pl.ds(i * _BI, _BI), :] = af.astype(jnp.bfloat16)

        @pl.when(s == 1)
        def _init():
            m_s[...] = blk_max
            deg_s[...] = blk_deg

        @pl.when(s > 1)
        def _acc():
            m_s[...] = jnp.maximum(m_s[...], blk_max)
            deg_s[...] += blk_deg

    @pl.when((s >= _NB + 1) & (s <= 2 * _NB))
    def _p2():
        i = s - (_NB + 1)
        af = a_s[pl.ds(i * _BI, _BI), :].astype(jnp.float32)
        hi = h_s[pl.ds(i * _BI, _BI), :]
        sblk = jax.lax.dot_general(
            hi, h_s[...], (((1,), (1,)), ((), ())),
            preferred_element_type=jnp.float32, precision=_HIGHEST,
        )
        e = af * jnp.exp(sblk - m_s[...])
        e_s[pl.ds(i * _BI, _BI), :] = e.astype(jnp.bfloat16)
        blk_deno = jnp.sum(e, axis=0, keepdims=True)
        deg_i = deg_s[0:1, pl.ds(i * _BI, _BI)]
        dinv_i = jnp.transpose(1.0 / jnp.sqrt(deg_i + 1.0), (1, 0))
        y = dinv_i * xw_s[pl.ds(i * _BI, _BI), :]
        blk_u = jax.lax.dot_general(
            af, y, (((0,), (0,)), ((), ())),
            preferred_element_type=jnp.float32,
        )

        @pl.when(s == _NB + 1)
        def _init():
            deno_s[...] = blk_deno
            u_s[...] = blk_u

        @pl.when(s > _NB + 1)
        def _acc():
            deno_s[...] += blk_deno
            u_s[...] += blk_u

    @pl.when((s >= 2 * _NB + 1) & (s <= 3 * _NB))
    def _p3():
        i = s - (2 * _NB + 1)
        e = e_s[pl.ds(i * _BI, _BI), :].astype(jnp.float32)
        deno_i = deno_s[0:1, pl.ds(i * _BI, _BI)]
        w_i = jnp.transpose(1.0 / (deno_i + 1e-8), (1, 0))
        blk_z = jnp.sum(w_i * e, axis=0, keepdims=True)

        @pl.when(s == 2 * _NB + 1)
        def _init():
            zs_s[...] = blk_z

        @pl.when(s > 2 * _NB + 1)
        def _acc():
            zs_s[...] += blk_z

    @pl.when(s == 3 * _NB + 1)
    def _p4():
        dinv = jnp.transpose(1.0 / jnp.sqrt(deg_s[...] + 1.0), (1, 0))
        xw = xw_s[...]
        agg = dinv * u_s[...] + (dinv * dinv) * xw
        rep = h_s[...] + jnp.maximum(agg + bg_ref[...], 0.0)
        rep_ref[...] = rep
        t_col = t_ref[...]
        z_col = t_col * jnp.transpose(zs_s[...], (1, 0))
        z_ref[...] = z_col
        hid = jnp.maximum(
            jnp.dot(rep, w1a_ref[...], preferred_element_type=jnp.float32,
                    precision=_HIGHEST)
            + t_col * w1t_ref[...]
            + z_col * w1z_ref[...]
            + b1_ref[...],
            0.0,
        )
        out_ref[...] = (
            jnp.dot(hid, w2_ref[...], preferred_element_type=jnp.float32,
                    precision=_HIGHEST)
            + b2_ref[...]
        )


@jax.jit
def kernel(x, A, t, W_emb, b_emb, W_gcn, b_gcn, W1, b1, W2, b2):
    n, n_in = x.shape
    n_h = W_emb.shape[1]
    f32 = jnp.float32

    const = lambda shape: pl.BlockSpec(shape, lambda s: (0, 0))
    a_spec = pl.BlockSpec(
        (_BI, n), lambda s: (jnp.clip(s - 1, 0, _NB - 1), 0))

    out, rep, z_col = pl.pallas_call(
        _mega_body,
        grid=(3 * _NB + 2,),
        in_specs=[
            const((n, n_in)),
            a_spec,
            const((n, 1)),
            const((n_in, n_h)),
            const((1, n_h)),
            const((n_h, n_h)),
            const((1, n_h)),
            const((n_h, n_h)),
            const((1, n_h)),
            const((1, n_h)),
            const((1, n_h)),
            const((n_h, 1)),
            const((1, 1)),
        ],
        out_specs=(const((n, 1)), const((n, n_h)), const((n, 1))),
        out_shape=(
            jax.ShapeDtypeStruct((n, 1), f32),
            jax.ShapeDtypeStruct((n, n_h), f32),
            jax.ShapeDtypeStruct((n, 1), f32),
        ),
        scratch_shapes=[
            pltpu.VMEM((n, n_h), f32),
            pltpu.VMEM((n, n_h), f32),
            pltpu.VMEM((n, n), jnp.bfloat16),
            pltpu.VMEM((n, n), jnp.bfloat16),
            pltpu.VMEM((1, 1), f32),
            pltpu.VMEM((1, n), f32),
            pltpu.VMEM((1, n), f32),
            pltpu.VMEM((n, n_h), f32),
            pltpu.VMEM((1, n), f32),
        ],
    )(x, A, t.reshape(n, 1), W_emb, b_emb.reshape(1, -1), W_gcn,
      b_gcn.reshape(1, -1), W1[:n_h], W1[n_h:n_h + 1], W1[n_h + 1:n_h + 2],
      b1.reshape(1, -1), W2, b2.reshape(1, 1))

    return out, rep, z_col.reshape(n)


# speedup vs baseline: 3504.3210x; 1.0037x over previous
import functools

import jax
import jax.numpy as jnp
from jax.experimental import pallas as pl
from jax.experimental.pallas import tpu as pltpu

_BI = 256
_NB = 8
_HIGHEST = jax.lax.Precision.HIGHEST


def _mega_body(x_ref, a_ref, t_ref, we_ref, be_ref, wg_ref, bg_ref,
               w1a_ref, w1t_ref, w1z_ref, b1_ref, w2_ref, b2_ref,
               out_ref, rep_ref, z_ref,
               h_s, xw_s, a_s, e_s, m_s, deg_s, deno_s, u_s, zs_s):
    s = pl.program_id(0)

    @pl.when(s == 0)
    def _p0():
        h = jnp.maximum(
            jnp.dot(x_ref[...], we_ref[...],
                    preferred_element_type=jnp.float32) + be_ref[...],
            0.0,
        )
        h_s[...] = h
        xw_s[...] = jnp.dot(h, wg_ref[...],
                            preferred_element_type=jnp.float32)

    @pl.when((s >= 1) & (s <= _NB))
    def _p1():
        i = s - 1
        a_blk = a_ref[...]
        af = a_blk.astype(jnp.float32)
        hi = h_s[pl.ds(i * _BI, _BI), :]
        sblk = jax.lax.dot_general(
            hi, h_s[...], (((1,), (1,)), ((), ())),
            preferred_element_type=jnp.float32, precision=_HIGHEST,
        )
        blk_max = jnp.max(jnp.where(a_blk != 0, sblk, -jnp.inf)).reshape(1, 1)
        blk_deg = jnp.sum(af, axis=0, keepdims=True)
        a_s[pl.ds(i * _BI, _BI), :] = af.astype(jnp.bfloat16)

        @pl.when(s == 1)
        def _init():
            m_s[...] = blk_max
            deg_s[...] = blk_deg

        @pl.when(s > 1)
        def _acc():
            m_s[...] = jnp.maximum(m_s[...], blk_max)
            deg_s[...] += blk_deg

    @pl.when((s >= _NB + 1) & (s <= 2 * _NB))
    def _p2():
        i = s - (_NB + 1)
        af = a_s[pl.ds(i * _BI, _BI), :].astype(jnp.float32)
        hi = h_s[pl.ds(i * _BI, _BI), :]
        sblk = jax.lax.dot_general(
            hi, h_s[...], (((1,), (1,)), ((), ())),
            preferred_element_type=jnp.float32, precision=_HIGHEST,
        )
        e = af * jnp.exp(sblk - m_s[...])
        e_s[pl.ds(i * _BI, _BI), :] = e
        blk_deno = jnp.sum(e, axis=0, keepdims=True)
        deg_i = deg_s[0:1, pl.ds(i * _BI, _BI)]
        dinv_i = jnp.transpose(1.0 / jnp.sqrt(deg_i + 1.0), (1, 0))
        y = dinv_i * xw_s[pl.ds(i * _BI, _BI), :]
        blk_u = jax.lax.dot_general(
            af, y, (((0,), (0,)), ((), ())),
            preferred_element_type=jnp.float32,
        )

        @pl.when(s == _NB + 1)
        def _init():
            deno_s[...] = blk_deno
            u_s[...] = blk_u

        @pl.when(s > _NB + 1)
        def _acc():
            deno_s[...] += blk_deno
            u_s[...] += blk_u

    @pl.when((s >= 2 * _NB + 1) & (s <= 3 * _NB))
    def _p3():
        i = s - (2 * _NB + 1)
        e = e_s[pl.ds(i * _BI, _BI), :]
        deno_i = deno_s[0:1, pl.ds(i * _BI, _BI)]
        w_i = jnp.transpose(1.0 / (deno_i + 1e-8), (1, 0))
        blk_z = jnp.sum(w_i * e, axis=0, keepdims=True)

        @pl.when(s == 2 * _NB + 1)
        def _init():
            zs_s[...] = blk_z

        @pl.when(s > 2 * _NB + 1)
        def _acc():
            zs_s[...] += blk_z

    @pl.when(s == 3 * _NB + 1)
    def _p4():
        dinv = jnp.transpose(1.0 / jnp.sqrt(deg_s[...] + 1.0), (1, 0))
        xw = xw_s[...]
        agg = dinv * u_s[...] + (dinv * dinv) * xw
        rep = h_s[...] + jnp.maximum(agg + bg_ref[...], 0.0)
        rep_ref[...] = rep
        t_col = t_ref[...]
        z_col = t_col * jnp.transpose(zs_s[...], (1, 0))
        z_ref[...] = z_col
        hid = jnp.maximum(
            jnp.dot(rep, w1a_ref[...], preferred_element_type=jnp.float32,
                    precision=_HIGHEST)
            + t_col * w1t_ref[...]
            + z_col * w1z_ref[...]
            + b1_ref[...],
            0.0,
        )
        out_ref[...] = (
            jnp.dot(hid, w2_ref[...], preferred_element_type=jnp.float32,
                    precision=_HIGHEST)
            + b2_ref[...]
        )


@jax.jit
def kernel(x, A, t, W_emb, b_emb, W_gcn, b_gcn, W1, b1, W2, b2):
    n, n_in = x.shape
    n_h = W_emb.shape[1]
    f32 = jnp.float32

    const = lambda shape: pl.BlockSpec(shape, lambda s: (0, 0))
    a_spec = pl.BlockSpec(
        (_BI, n), lambda s: (jnp.clip(s - 1, 0, _NB - 1), 0))

    out, rep, z_col = pl.pallas_call(
        _mega_body,
        grid=(3 * _NB + 2,),
        in_specs=[
            const((n, n_in)),
            a_spec,
            const((n, 1)),
            const((n_in, n_h)),
            const((1, n_h)),
            const((n_h, n_h)),
            const((1, n_h)),
            const((n_h, n_h)),
            const((1, n_h)),
            const((1, n_h)),
            const((1, n_h)),
            const((n_h, 1)),
            const((1, 1)),
        ],
        out_specs=(const((n, 1)), const((n, n_h)), const((n, 1))),
        out_shape=(
            jax.ShapeDtypeStruct((n, 1), f32),
            jax.ShapeDtypeStruct((n, n_h), f32),
            jax.ShapeDtypeStruct((n, 1), f32),
        ),
        scratch_shapes=[
            pltpu.VMEM((n, n_h), f32),
            pltpu.VMEM((n, n_h), f32),
            pltpu.VMEM((n, n), jnp.bfloat16),
            pltpu.VMEM((n, n), f32),
            pltpu.VMEM((1, 1), f32),
            pltpu.VMEM((1, n), f32),
            pltpu.VMEM((1, n), f32),
            pltpu.VMEM((n, n_h), f32),
            pltpu.VMEM((1, n), f32),
        ],
    )(x, A, t.reshape(n, 1), W_emb, b_emb.reshape(1, -1), W_gcn,
      b_gcn.reshape(1, -1), W1[:n_h], W1[n_h:n_h + 1], W1[n_h + 1:n_h + 2],
      b1.reshape(1, -1), W2, b2.reshape(1, 1))

    return out, rep, z_col.reshape(n)


# speedup vs baseline: 4124.3724x; 1.1769x over previous
import jax
import jax.numpy as jnp
from jax.experimental import pallas as pl
from jax.experimental.pallas import tpu as pltpu

_BI = 512
_NB = 4
_HIGHEST = jax.lax.Precision.HIGHEST


def _mega_body(x_ref, a_ref, t_ref, we_ref, be_ref, wg_ref, bg_ref,
               w1a_ref, w1t_ref, w1z_ref, b1_ref, w2_ref, b2_ref,
               out_ref, rep_ref, z_ref,
               h_s, xw_s, a_s, e_s, mm_s, mf_s, c_s, deg_s, csum_s,
               w_s, dinv_s, u_s, zs_s):
    s = pl.program_id(0)

    @pl.when(s == 0)
    def _p0():
        h = jnp.maximum(
            jnp.dot(x_ref[...], we_ref[...],
                    preferred_element_type=jnp.float32) + be_ref[...],
            0.0,
        )
        h_s[...] = h
        xw_s[...] = jnp.dot(h, wg_ref[...],
                            preferred_element_type=jnp.float32)

    @pl.when((s >= 1) & (s <= _NB))
    def _p1():
        b = s - 1
        a_blk = a_ref[...]
        af = a_blk.astype(jnp.float32)
        hi = h_s[pl.ds(b * _BI, _BI), :]
        sblk = jax.lax.dot_general(
            hi, h_s[...], (((1,), (1,)), ((), ())),
            preferred_element_type=jnp.float32, precision=_HIGHEST,
        )
        m_full = jnp.max(sblk).reshape(1, 1)
        m_mask = jnp.max(jnp.where(a_blk != 0, sblk, -jnp.inf)).reshape(1, 1)
        e = af * jnp.exp(sblk - m_full)
        e_s[pl.ds(b * _BI, _BI), :] = e
        csum_s[pl.ds(b, 1), :] = jnp.sum(e, axis=0, keepdims=True)
        mf_s[pl.ds(b, 1), 0:1] = m_full
        mm_s[pl.ds(b, 1), 0:1] = m_mask
        a_s[pl.ds(b * _BI, _BI), :] = af.astype(jnp.bfloat16)
        blk_deg = jnp.sum(af, axis=0, keepdims=True)

        @pl.when(s == 1)
        def _init():
            deg_s[...] = blk_deg

        @pl.when(s > 1)
        def _acc():
            deg_s[...] += blk_deg

    @pl.when(s == _NB + 1)
    def _mid():
        m = jnp.max(mm_s[:, 0:1]).reshape(1, 1)
        c = jnp.exp(jnp.minimum(mf_s[:, 0:1] - m, 80.0))
        c_s[...] = c
        deno = jnp.sum(c * csum_s[...], axis=0, keepdims=True)
        w_s[...] = 1.0 / (deno + 1e-8)
        dinv_s[...] = 1.0 / jnp.sqrt(deg_s[...] + 1.0)

    @pl.when((s >= _NB + 2) & (s <= 2 * _NB + 1))
    def _p2():
        b = s - (_NB + 2)
        e = e_s[pl.ds(b * _BI, _BI), :]
        c_b = c_s[pl.ds(b, 1), 0:1]
        w_i = jnp.transpose(w_s[0:1, pl.ds(b * _BI, _BI)], (1, 0))
        blk_z = jnp.sum((c_b * w_i) * e, axis=0, keepdims=True)
        af = a_s[pl.ds(b * _BI, _BI), :].astype(jnp.float32)
        dinv_i = jnp.transpose(dinv_s[0:1, pl.ds(b * _BI, _BI)], (1, 0))
        y = dinv_i * xw_s[pl.ds(b * _BI, _BI), :]
        blk_u = jax.lax.dot_general(
            af, y, (((0,), (0,)), ((), ())),
            preferred_element_type=jnp.float32,
        )

        @pl.when(s == _NB + 2)
        def _init():
            zs_s[...] = blk_z
            u_s[...] = blk_u

        @pl.when(s > _NB + 2)
        def _acc():
            zs_s[...] += blk_z
            u_s[...] += blk_u

    @pl.when(s == 2 * _NB + 2)
    def _p4():
        dinv = jnp.transpose(dinv_s[...], (1, 0))
        xw = xw_s[...]
        agg = dinv * u_s[...] + (dinv * dinv) * xw
        rep = h_s[...] + jnp.maximum(agg + bg_ref[...], 0.0)
        rep_ref[...] = rep
        t_col = t_ref[...]
        z_col = t_col * jnp.transpose(zs_s[...], (1, 0))
        z_ref[...] = z_col
        hid = jnp.maximum(
            jnp.dot(rep, w1a_ref[...], preferred_element_type=jnp.float32,
                    precision=_HIGHEST)
            + t_col * w1t_ref[...]
            + z_col * w1z_ref[...]
            + b1_ref[...],
            0.0,
        )
        out_ref[...] = (
            jnp.dot(hid, w2_ref[...], preferred_element_type=jnp.float32,
                    precision=_HIGHEST)
            + b2_ref[...]
        )


@jax.jit
def kernel(x, A, t, W_emb, b_emb, W_gcn, b_gcn, W1, b1, W2, b2):
    n, n_in = x.shape
    n_h = W_emb.shape[1]
    f32 = jnp.float32

    const = lambda shape: pl.BlockSpec(shape, lambda s: (0, 0))
    a_spec = pl.BlockSpec(
        (_BI, n), lambda s: (jnp.clip(s - 1, 0, _NB - 1), 0))

    out, rep, z_col = pl.pallas_call(
        _mega_body,
        grid=(2 * _NB + 3,),
        in_specs=[
            const((n, n_in)),
            a_spec,
            const((n, 1)),
            const((n_in, n_h)),
            const((1, n_h)),
            const((n_h, n_h)),
            const((1, n_h)),
            const((n_h, n_h)),
            const((1, n_h)),
            const((1, n_h)),
            const((1, n_h)),
            const((n_h, 1)),
            const((1, 1)),
        ],
        out_specs=(const((n, 1)), const((n, n_h)), const((n, 1))),
        out_shape=(
            jax.ShapeDtypeStruct((n, 1), f32),
            jax.ShapeDtypeStruct((n, n_h), f32),
            jax.ShapeDtypeStruct((n, 1), f32),
        ),
        scratch_shapes=[
            pltpu.VMEM((n, n_h), f32),
            pltpu.VMEM((n, n_h), f32),
            pltpu.VMEM((n, n), jnp.bfloat16),
            pltpu.VMEM((n, n), f32),
            pltpu.VMEM((_NB, 128), f32),
            pltpu.VMEM((_NB, 128), f32),
            pltpu.VMEM((_NB, 1), f32),
            pltpu.VMEM((1, n), f32),
            pltpu.VMEM((_NB, n), f32),
            pltpu.VMEM((1, n), f32),
            pltpu.VMEM((1, n), f32),
            pltpu.VMEM((n, n_h), f32),
            pltpu.VMEM((1, n), f32),
        ],
    )(x, A, t.reshape(n, 1), W_emb, b_emb.reshape(1, -1), W_gcn,
      b_gcn.reshape(1, -1), W1[:n_h], W1[n_h:n_h + 1], W1[n_h + 1:n_h + 2],
      b1.reshape(1, -1), W2, b2.reshape(1, 1))

    return out, rep, z_col.reshape(n)


# speedup vs baseline: 4161.6349x; 1.0090x over previous
import jax
import jax.numpy as jnp
from jax.experimental import pallas as pl
from jax.experimental.pallas import tpu as pltpu

_BI = 512
_NB = 4
_HIGHEST = jax.lax.Precision.HIGHEST


def _mega_body(x_ref, a_ref, t_ref, we_ref, be_ref, wg_ref, bg_ref,
               w1a_ref, w1t_ref, w1z_ref, b1_ref, w2_ref, b2_ref,
               out_ref, rep_ref, z_ref,
               h_s, xw_s, a_s, e_s, mm_s, c_s, deg_s, csum_s,
               w_s, dinv_s, u_s, zs_s):
    s = pl.program_id(0)

    @pl.when(s == 0)
    def _p0():
        h = jnp.maximum(
            jnp.dot(x_ref[...], we_ref[...],
                    preferred_element_type=jnp.float32) + be_ref[...],
            0.0,
        )
        h_s[...] = h
        xw_s[...] = jnp.dot(h, wg_ref[...],
                            preferred_element_type=jnp.float32)

    @pl.when((s >= 1) & (s <= _NB))
    def _p1():
        b = s - 1
        a_blk = a_ref[...]
        af = a_blk.astype(jnp.float32)
        hi = h_s[pl.ds(b * _BI, _BI), :]
        sblk = jax.lax.dot_general(
            hi, h_s[...], (((1,), (1,)), ((), ())),
            preferred_element_type=jnp.float32, precision=_HIGHEST,
        )
        mask = a_blk != 0
        m_mask = jnp.max(jnp.where(mask, sblk, -jnp.inf)).reshape(1, 1)
        e = jnp.where(mask, jnp.exp(sblk - m_mask), 0.0)
        e_s[pl.ds(b * _BI, _BI), :] = e
        csum_s[pl.ds(b, 1), :] = jnp.sum(e, axis=0, keepdims=True)
        mm_s[pl.ds(b, 1), 0:1] = m_mask
        a_s[pl.ds(b * _BI, _BI), :] = af.astype(jnp.bfloat16)
        blk_deg = jnp.sum(af, axis=0, keepdims=True)

        @pl.when(s == 1)
        def _init():
            deg_s[...] = blk_deg

        @pl.when(s > 1)
        def _acc():
            deg_s[...] += blk_deg

    @pl.when(s == _NB + 1)
    def _mid():
        m = jnp.maximum(jnp.max(mm_s[:, 0:1]), -1e38).reshape(1, 1)
        c = jnp.exp(mm_s[:, 0:1] - m)

        c_s[...] = c
        deno = jnp.sum(c * csum_s[...], axis=0, keepdims=True)
        w_s[...] = 1.0 / (deno + 1e-8)
        dinv_s[...] = 1.0 / jnp.sqrt(deg_s[...] + 1.0)

    @pl.when((s >= _NB + 2) & (s <= 2 * _NB + 1))
    def _p2():
        b = s - (_NB + 2)
        e = e_s[pl.ds(b * _BI, _BI), :]
        c_b = c_s[pl.ds(b, 1), 0:1]
        w_i = jnp.transpose(w_s[0:1, pl.ds(b * _BI, _BI)], (1, 0))
        blk_z = jnp.sum((c_b * w_i) * e, axis=0, keepdims=True)
        a_bf = a_s[pl.ds(b * _BI, _BI), :]
        dinv_i = jnp.transpose(dinv_s[0:1, pl.ds(b * _BI, _BI)], (1, 0))
        y = (dinv_i * xw_s[pl.ds(b * _BI, _BI), :]).astype(jnp.bfloat16)
        blk_u = jax.lax.dot_general(
            a_bf, y, (((0,), (0,)), ((), ())),
            preferred_element_type=jnp.float32,
        )

        @pl.when(s == _NB + 2)
        def _init():
            zs_s[...] = blk_z
            u_s[...] = blk_u

        @pl.when(s > _NB + 2)
        def _acc():
            zs_s[...] += blk_z
            u_s[...] += blk_u

    @pl.when(s == 2 * _NB + 2)
    def _p4():
        dinv = jnp.transpose(dinv_s[...], (1, 0))
        xw = xw_s[...]
        agg = dinv * u_s[...] + (dinv * dinv) * xw
        rep = h_s[...] + jnp.maximum(agg + bg_ref[...], 0.0)
        rep_ref[...] = rep
        t_col = t_ref[...]
        z_col = t_col * jnp.transpose(zs_s[...], (1, 0))
        z_ref[...] = z_col
        hid = jnp.maximum(
            jnp.dot(rep, w1a_ref[...], preferred_element_type=jnp.float32,
                    precision=_HIGHEST)
            + t_col * w1t_ref[...]
            + z_col * w1z_ref[...]
            + b1_ref[...],
            0.0,
        )
        out_ref[...] = (
            jnp.dot(hid, w2_ref[...], preferred_element_type=jnp.float32,
                    precision=_HIGHEST)
            + b2_ref[...]
        )


@jax.jit
def kernel(x, A, t, W_emb, b_emb, W_gcn, b_gcn, W1, b1, W2, b2):
    n, n_in = x.shape
    n_h = W_emb.shape[1]
    f32 = jnp.float32

    const = lambda shape: pl.BlockSpec(shape, lambda s: (0, 0))
    a_spec = pl.BlockSpec(
        (_BI, n), lambda s: (jnp.clip(s - 1, 0, _NB - 1), 0))

    out, rep, z_col = pl.pallas_call(
        _mega_body,
        grid=(2 * _NB + 3,),
        in_specs=[
            const((n, n_in)),
            a_spec,
            const((n, 1)),
            const((n_in, n_h)),
            const((1, n_h)),
            const((n_h, n_h)),
            const((1, n_h)),
            const((n_h, n_h)),
            const((1, n_h)),
            const((1, n_h)),
            const((1, n_h)),
            const((n_h, 1)),
            const((1, 1)),
        ],
        out_specs=(const((n, 1)), const((n, n_h)), const((n, 1))),
        out_shape=(
            jax.ShapeDtypeStruct((n, 1), f32),
            jax.ShapeDtypeStruct((n, n_h), f32),
            jax.ShapeDtypeStruct((n, 1), f32),
        ),
        scratch_shapes=[
            pltpu.VMEM((n, n_h), f32),
            pltpu.VMEM((n, n_h), f32),
            pltpu.VMEM((n, n), jnp.bfloat16),
            pltpu.VMEM((n, n), f32),
            pltpu.VMEM((_NB, 128), f32),
            pltpu.VMEM((_NB, 1), f32),
            pltpu.VMEM((1, n), f32),
            pltpu.VMEM((_NB, n), f32),
            pltpu.VMEM((1, n), f32),
            pltpu.VMEM((1, n), f32),
            pltpu.VMEM((n, n_h), f32),
            pltpu.VMEM((1, n), f32),
        ],
    )(x, A, t.reshape(n, 1), W_emb, b_emb.reshape(1, -1), W_gcn,
      b_gcn.reshape(1, -1), W1[:n_h], W1[n_h:n_h + 1], W1[n_h + 1:n_h + 2],
      b1.reshape(1, -1), W2, b2.reshape(1, 1))

    return out, rep, z_col.reshape(n)


# speedup vs baseline: 4354.7302x; 1.0464x over previous
import jax
import jax.numpy as jnp
from jax.experimental import pallas as pl
from jax.experimental.pallas import tpu as pltpu

_BI = 512
_NB = 4
_HIGHEST = jax.lax.Precision.HIGHEST


def _mega_body(x_ref, a_ref, t_ref, we_ref, be_ref, wg_ref, bg_ref,
               w1a_ref, w1t_ref, w1z_ref, b1_ref, w2_ref, b2_ref,
               out_ref, rep_ref, z_ref,
               h_s, xw_s, a_s, e_s, mm_s, c_s, deg_s, csum_s,
               w_s, dinv_s, u_s, zs_s):
    s = pl.program_id(0)

    @pl.when(s == 0)
    def _p0():
        h = jnp.maximum(
            jnp.dot(x_ref[...], we_ref[...],
                    preferred_element_type=jnp.float32) + be_ref[...],
            0.0,
        )
        h_s[...] = h
        xw_s[...] = jnp.dot(h, wg_ref[...],
                            preferred_element_type=jnp.float32)

    @pl.when((s >= 1) & (s <= _NB))
    def _p1():
        b = s - 1
        a_blk = a_ref[...]
        af = a_blk.astype(jnp.float32)
        hi = h_s[pl.ds(b * _BI, _BI), :]
        sblk = jax.lax.dot_general(
            hi, h_s[...], (((1,), (1,)), ((), ())),
            preferred_element_type=jnp.float32, precision=_HIGHEST,
        )
        sm = jnp.where(a_blk != 0, sblk, -jnp.inf)
        m_mask = jnp.max(sm).reshape(1, 1)
        e = jnp.exp(sm - jnp.maximum(m_mask, -1e38))
        e_s[pl.ds(b * _BI, _BI), :] = e
        ones_r = jnp.ones((1, _BI), jnp.float32)
        csum_s[pl.ds(b, 1), :] = jax.lax.dot_general(
            ones_r, e, (((1,), (0,)), ((), ())),
            preferred_element_type=jnp.float32,
        )
        mm_s[pl.ds(b, 1), 0:1] = m_mask
        a_s[pl.ds(b * _BI, _BI), :] = af.astype(jnp.bfloat16)
        blk_deg = jax.lax.dot_general(
            ones_r, af, (((1,), (0,)), ((), ())),
            preferred_element_type=jnp.float32,
        )

        @pl.when(s == 1)
        def _init():
            deg_s[...] = blk_deg

        @pl.when(s > 1)
        def _acc():
            deg_s[...] += blk_deg

    @pl.when(s == _NB + 1)
    def _mid():
        m = jnp.maximum(jnp.max(mm_s[:, 0:1]), -1e38).reshape(1, 1)
        c = jnp.exp(mm_s[:, 0:1] - m)

        c_s[...] = c
        deno = jnp.sum(c * csum_s[...], axis=0, keepdims=True)
        w_s[...] = 1.0 / (deno + 1e-8)
        dinv_s[...] = 1.0 / jnp.sqrt(deg_s[...] + 1.0)

    @pl.when((s >= _NB + 2) & (s <= 2 * _NB + 1))
    def _p2():
        b = s - (_NB + 2)
        e = e_s[pl.ds(b * _BI, _BI), :]
        c_b = c_s[pl.ds(b, 1), 0:1]
        wc = c_b * w_s[0:1, pl.ds(b * _BI, _BI)]
        blk_z = jax.lax.dot_general(
            wc, e, (((1,), (0,)), ((), ())),
            preferred_element_type=jnp.float32,
        )
        a_bf = a_s[pl.ds(b * _BI, _BI), :]
        dinv_i = jnp.transpose(dinv_s[0:1, pl.ds(b * _BI, _BI)], (1, 0))
        y = (dinv_i * xw_s[pl.ds(b * _BI, _BI), :]).astype(jnp.bfloat16)
        blk_u = jax.lax.dot_general(
            a_bf, y, (((0,), (0,)), ((), ())),
            preferred_element_type=jnp.float32,
        )

        @pl.when(s == _NB + 2)
        def _init():
            zs_s[...] = blk_z
            u_s[...] = blk_u

        @pl.when(s > _NB + 2)
        def _acc():
            zs_s[...] += blk_z
            u_s[...] += blk_u

    @pl.when(s == 2 * _NB + 2)
    def _p4():
        dinv = jnp.transpose(dinv_s[...], (1, 0))
        xw = xw_s[...]
        agg = dinv * u_s[...] + (dinv * dinv) * xw
        rep = h_s[...] + jnp.maximum(agg + bg_ref[...], 0.0)
        rep_ref[...] = rep
        t_col = t_ref[...]
        z_col = t_col * jnp.transpose(zs_s[...], (1, 0))
        z_ref[...] = z_col
        hid = jnp.maximum(
            jnp.dot(rep, w1a_ref[...], preferred_element_type=jnp.float32,
                    precision=_HIGHEST)
            + t_col * w1t_ref[...]
            + z_col * w1z_ref[...]
            + b1_ref[...],
            0.0,
        )
        out_ref[...] = (
            jnp.dot(hid, w2_ref[...], preferred_element_type=jnp.float32,
                    precision=_HIGHEST)
            + b2_ref[...]
        )


@jax.jit
def kernel(x, A, t, W_emb, b_emb, W_gcn, b_gcn, W1, b1, W2, b2):
    n, n_in = x.shape
    n_h = W_emb.shape[1]
    f32 = jnp.float32

    const = lambda shape: pl.BlockSpec(shape, lambda s: (0, 0))
    a_spec = pl.BlockSpec(
        (_BI, n), lambda s: (jnp.clip(s - 1, 0, _NB - 1), 0))

    out, rep, z_col = pl.pallas_call(
        _mega_body,
        grid=(2 * _NB + 3,),
        in_specs=[
            const((n, n_in)),
            a_spec,
            const((n, 1)),
            const((n_in, n_h)),
            const((1, n_h)),
            const((n_h, n_h)),
            const((1, n_h)),
            const((n_h, n_h)),
            const((1, n_h)),
            const((1, n_h)),
            const((1, n_h)),
            const((n_h, 1)),
            const((1, 1)),
        ],
        out_specs=(const((n, 1)), const((n, n_h)), const((n, 1))),
        out_shape=(
            jax.ShapeDtypeStruct((n, 1), f32),
            jax.ShapeDtypeStruct((n, n_h), f32),
            jax.ShapeDtypeStruct((n, 1), f32),
        ),
        scratch_shapes=[
            pltpu.VMEM((n, n_h), f32),
            pltpu.VMEM((n, n_h), f32),
            pltpu.VMEM((n, n), jnp.bfloat16),
            pltpu.VMEM((n, n), f32),
            pltpu.VMEM((_NB, 128), f32),
            pltpu.VMEM((_NB, 1), f32),
            pltpu.VMEM((1, n), f32),
            pltpu.VMEM((_NB, n), f32),
            pltpu.VMEM((1, n), f32),
            pltpu.VMEM((1, n), f32),
            pltpu.VMEM((n, n_h), f32),
            pltpu.VMEM((1, n), f32),
        ],
    )(x, A, t.reshape(n, 1), W_emb, b_emb.reshape(1, -1), W_gcn,
      b_gcn.reshape(1, -1), W1[:n_h], W1[n_h:n_h + 1], W1[n_h + 1:n_h + 2],
      b1.reshape(1, -1), W2, b2.reshape(1, 1))

    return out, rep, z_col.reshape(n)


# speedup vs baseline: 4418.1325x; 1.0146x over previous
import jax
import jax.numpy as jnp
from jax.experimental import pallas as pl
from jax.experimental.pallas import tpu as pltpu

_BI = 512
_NB = 4
_HIGHEST = jax.lax.Precision.HIGHEST


def _mega_body(x_ref, a_ref, t_ref, we_ref, be_ref, wg_ref, bg_ref,
               w1a_ref, w1t_ref, w1z_ref, b1_ref, w2_ref, b2_ref,
               out_ref, rep_ref, z_ref,
               h_s, xw_s, a_s, e_s, mm_s, deg_s, csum_s):
    s = pl.program_id(0)

    @pl.when(s == 0)
    def _p0():
        h = jnp.maximum(
            jnp.dot(x_ref[...], we_ref[...],
                    preferred_element_type=jnp.float32) + be_ref[...],
            0.0,
        )
        h_s[...] = h
        xw_s[...] = jnp.dot(h, wg_ref[...],
                            preferred_element_type=jnp.float32)

    @pl.when((s >= 1) & (s <= _NB))
    def _p1():
        b = s - 1
        a_blk = a_ref[...]
        hi = h_s[pl.ds(b * _BI, _BI), :]
        sblk = jax.lax.dot_general(
            hi, h_s[...], (((1,), (1,)), ((), ())),
            preferred_element_type=jnp.float32, precision=_HIGHEST,
        )
        sm = jnp.where(a_blk != 0, sblk, -jnp.inf)
        m_mask = jnp.max(sm).reshape(1, 1)
        e = jnp.exp(sm - jnp.maximum(m_mask, -1e38))
        e_s[pl.ds(b * _BI, _BI), :] = e
        ones_r = jnp.ones((1, _BI), jnp.float32)
        csum_s[pl.ds(b, 1), :] = jax.lax.dot_general(
            ones_r, e, (((1,), (0,)), ((), ())),
            preferred_element_type=jnp.float32,
        )
        mm_s[pl.ds(b, 1), 0:1] = m_mask
        af = a_blk.astype(jnp.float32)
        a_s[pl.ds(b * _BI, _BI), :] = af.astype(jnp.bfloat16)
        blk_deg = jax.lax.dot_general(
            ones_r, af, (((1,), (0,)), ((), ())),
            preferred_element_type=jnp.float32,
        )

        @pl.when(s == 1)
        def _init():
            deg_s[...] = blk_deg

        @pl.when(s > 1)
        def _acc():
            deg_s[...] += blk_deg

    @pl.when(s == _NB + 1)
    def _fin():
        m = jnp.maximum(jnp.max(mm_s[:, 0:1]), -1e38).reshape(1, 1)
        c = jnp.exp(mm_s[:, 0:1] - m)
        deno = jnp.sum(c * csum_s[...], axis=0, keepdims=True)
        w = 1.0 / (deno + 1e-8)
        wc = jnp.concatenate(
            [c[b:b + 1, 0:1] * w[:, b * _BI:(b + 1) * _BI]
             for b in range(_NB)], axis=1)
        zs = jax.lax.dot_general(
            wc, e_s[...], (((1,), (0,)), ((), ())),
            preferred_element_type=jnp.float32,
        )
        t_col = t_ref[...]
        z_col = t_col * jnp.transpose(zs, (1, 0))
        z_ref[...] = z_col
        dinv = 1.0 / jnp.sqrt(deg_s[...] + 1.0)
        dinv_col = jnp.transpose(dinv, (1, 0))
        xw = xw_s[...]
        y = (dinv_col * xw).astype(jnp.bfloat16)
        u = jax.lax.dot_general(
            a_s[...], y, (((0,), (0,)), ((), ())),
            preferred_element_type=jnp.float32,
        )
        agg = dinv_col * u + (dinv_col * dinv_col) * xw
        rep = h_s[...] + jnp.maximum(agg + bg_ref[...], 0.0)
        rep_ref[...] = rep
        hid = jnp.maximum(
            jnp.dot(rep, w1a_ref[...], preferred_element_type=jnp.float32,
                    precision=_HIGHEST)
            + t_col * w1t_ref[...]
            + z_col * w1z_ref[...]
            + b1_ref[...],
            0.0,
        )
        out_ref[...] = (
            jnp.dot(hid, w2_ref[...], preferred_element_type=jnp.float32,
                    precision=_HIGHEST)
            + b2_ref[...]
        )


@jax.jit
def kernel(x, A, t, W_emb, b_emb, W_gcn, b_gcn, W1, b1, W2, b2):
    n, n_in = x.shape
    n_h = W_emb.shape[1]
    f32 = jnp.float32

    const = lambda shape: pl.BlockSpec(shape, lambda s: (0, 0))
    a_spec = pl.BlockSpec(
        (_BI, n), lambda s: (jnp.clip(s - 1, 0, _NB - 1), 0))

    out, rep, z_col = pl.pallas_call(
        _mega_body,
        grid=(_NB + 2,),
        in_specs=[
            const((n, n_in)),
            a_spec,
            const((n, 1)),
            const((n_in, n_h)),
            const((1, n_h)),
            const((n_h, n_h)),
            const((1, n_h)),
            const((n_h, n_h)),
            const((1, n_h)),
            const((1, n_h)),
            const((1, n_h)),
            const((n_h, 1)),
            const((1, 1)),
        ],
        out_specs=(const((n, 1)), const((n, n_h)), const((n, 1))),
        out_shape=(
            jax.ShapeDtypeStruct((n, 1), f32),
            jax.ShapeDtypeStruct((n, n_h), f32),
            jax.ShapeDtypeStruct((n, 1), f32),
        ),
        scratch_shapes=[
            pltpu.VMEM((n, n_h), f32),
            pltpu.VMEM((n, n_h), f32),
            pltpu.VMEM((n, n), jnp.bfloat16),
            pltpu.VMEM((n, n), f32),
            pltpu.VMEM((_NB, 128), f32),
            pltpu.VMEM((1, n), f32),
            pltpu.VMEM((_NB, n), f32),
        ],
    )(x, A, t.reshape(n, 1), W_emb, b_emb.reshape(1, -1), W_gcn,
      b_gcn.reshape(1, -1), W1[:n_h], W1[n_h:n_h + 1], W1[n_h + 1:n_h + 2],
      b1.reshape(1, -1), W2, b2.reshape(1, 1))

    return out, rep, z_col.reshape(n)


# speedup vs baseline: 5425.8663x; 1.2281x over previous
import jax
import jax.numpy as jnp
from jax.experimental import pallas as pl
from jax.experimental.pallas import tpu as pltpu

_BI = 512
_NB = 4
_HIGHEST = jax.lax.Precision.HIGHEST


def _mega_body(x_ref, a_ref, t_ref, we_ref, be_ref, wg_ref, bg_ref,
               w1a_ref, w1t_ref, w1z_ref, b1_ref, w2_ref, b2_ref,
               out_ref, rep_ref, z_ref,
               h_s, xw_s, hl_s, hr_s, a_s, e_s, mm_s, deg_s, csum_s):
    s = pl.program_id(0)

    @pl.when(s == 0)
    def _p0():
        h = jnp.maximum(
            jnp.dot(x_ref[...], we_ref[...],
                    preferred_element_type=jnp.float32) + be_ref[...],
            0.0,
        )
        h_s[...] = h
        xw_s[...] = jnp.dot(h, wg_ref[...],
                            preferred_element_type=jnp.float32)
        h_hi = h.astype(jnp.bfloat16)
        h_lo = (h - h_hi.astype(jnp.float32)).astype(jnp.bfloat16)
        hl_s[...] = jnp.concatenate([h_hi, h_hi, h_lo], axis=1)
        hr_s[...] = jnp.concatenate([h_hi, h_lo, h_hi], axis=1)

    @pl.when((s >= 1) & (s <= _NB))
    def _p1():
        b = s - 1
        a_blk = a_ref[...]
        hi = hl_s[pl.ds(b * _BI, _BI), :]
        sblk = jax.lax.dot_general(
            hi, hr_s[...], (((1,), (1,)), ((), ())),
            preferred_element_type=jnp.float32,
        )
        sm = jnp.where(a_blk != 0, sblk, -jnp.inf)
        m_mask = jnp.max(sm).reshape(1, 1)
        e = jnp.exp(sm - jnp.maximum(m_mask, -1e38))
        e_s[pl.ds(b * _BI, _BI), :] = e
        ones_r = jnp.ones((1, _BI), jnp.float32)
        csum_s[pl.ds(b, 1), :] = jax.lax.dot_general(
            ones_r, e, (((1,), (0,)), ((), ())),
            preferred_element_type=jnp.float32,
        )
        mm_s[pl.ds(b, 1), 0:1] = m_mask
        af = a_blk.astype(jnp.float32)
        a_s[pl.ds(b * _BI, _BI), :] = af.astype(jnp.bfloat16)
        blk_deg = jax.lax.dot_general(
            ones_r, af, (((1,), (0,)), ((), ())),
            preferred_element_type=jnp.float32,
        )

        @pl.when(s == 1)
        def _init():
            deg_s[...] = blk_deg

        @pl.when(s > 1)
        def _acc():
            deg_s[...] += blk_deg

    @pl.when(s == _NB + 1)
    def _fin():
        m = jnp.maximum(jnp.max(mm_s[:, 0:1]), -1e38).reshape(1, 1)
        c = jnp.exp(mm_s[:, 0:1] - m)
        deno = jnp.sum(c * csum_s[...], axis=0, keepdims=True)
        w = 1.0 / (deno + 1e-8)
        wc = jnp.concatenate(
            [c[b:b + 1, 0:1] * w[:, b * _BI:(b + 1) * _BI]
             for b in range(_NB)], axis=1)
        zs = jax.lax.dot_general(
            wc, e_s[...], (((1,), (0,)), ((), ())),
            preferred_element_type=jnp.float32,
        )
        t_col = t_ref[...]
        z_col = t_col * jnp.transpose(zs, (1, 0))
        z_ref[...] = z_col
        dinv = 1.0 / jnp.sqrt(deg_s[...] + 1.0)
        dinv_col = jnp.transpose(dinv, (1, 0))
        xw = xw_s[...]
        y = (dinv_col * xw).astype(jnp.bfloat16)
        u = jax.lax.dot_general(
            a_s[...], y, (((0,), (0,)), ((), ())),
            preferred_element_type=jnp.float32,
        )
        agg = dinv_col * u + (dinv_col * dinv_col) * xw
        rep = h_s[...] + jnp.maximum(agg + bg_ref[...], 0.0)
        rep_ref[...] = rep
        hid = jnp.maximum(
            jnp.dot(rep, w1a_ref[...], preferred_element_type=jnp.float32,
                    precision=_HIGHEST)
            + t_col * w1t_ref[...]
            + z_col * w1z_ref[...]
            + b1_ref[...],
            0.0,
        )
        out_ref[...] = (
            jnp.dot(hid, w2_ref[...], preferred_element_type=jnp.float32,
                    precision=_HIGHEST)
            + b2_ref[...]
        )


@jax.jit
def kernel(x, A, t, W_emb, b_emb, W_gcn, b_gcn, W1, b1, W2, b2):
    n, n_in = x.shape
    n_h = W_emb.shape[1]
    f32 = jnp.float32

    const = lambda shape: pl.BlockSpec(shape, lambda s: (0, 0))
    a_spec = pl.BlockSpec(
        (_BI, n), lambda s: (jnp.clip(s - 1, 0, _NB - 1), 0))

    out, rep, z_col = pl.pallas_call(
        _mega_body,
        grid=(_NB + 2,),
        in_specs=[
            const((n, n_in)),
            a_spec,
            const((n, 1)),
            const((n_in, n_h)),
            const((1, n_h)),
            const((n_h, n_h)),
            const((1, n_h)),
            const((n_h, n_h)),
            const((1, n_h)),
            const((1, n_h)),
            const((1, n_h)),
            const((n_h, 1)),
            const((1, 1)),
        ],
        out_specs=(const((n, 1)), const((n, n_h)), const((n, 1))),
        out_shape=(
            jax.ShapeDtypeStruct((n, 1), f32),
            jax.ShapeDtypeStruct((n, n_h), f32),
            jax.ShapeDtypeStruct((n, 1), f32),
        ),
        scratch_shapes=[
            pltpu.VMEM((n, n_h), f32),
            pltpu.VMEM((n, n_h), f32),
            pltpu.VMEM((n, 3 * n_h), jnp.bfloat16),
            pltpu.VMEM((n, 3 * n_h), jnp.bfloat16),
            pltpu.VMEM((n, n), jnp.bfloat16),
            pltpu.VMEM((n, n), f32),
            pltpu.VMEM((_NB, 128), f32),
            pltpu.VMEM((1, n), f32),
            pltpu.VMEM((_NB, n), f32),
        ],
    )(x, A, t.reshape(n, 1), W_emb, b_emb.reshape(1, -1), W_gcn,
      b_gcn.reshape(1, -1), W1[:n_h], W1[n_h:n_h + 1], W1[n_h + 1:n_h + 2],
      b1.reshape(1, -1), W2, b2.reshape(1, 1))

    return out, rep, z_col.reshape(n)
